# baseline (speedup 1.0000x reference)
import jax
import jax.numpy as jnp
from jax import lax
from jax.experimental import pallas as pl
from jax.experimental.pallas import tpu as pltpu

B, H, W = 2, 224, 224
K, D = 19, 1024
P = H * W
KPAD = 32
DT = 128
PC = 3584
NS = 4


def _outer_body(src_ref, colors_ref, table_ref, out_hbm, onehot_ref):
    for b in range(B):
        q = (src_ref[b] * 127.5 + 127.5).astype(jnp.int32)
        match = None
        for c in range(3):
            eq = q[c:c + 1, :] == colors_ref[:, c:c + 1]
            match = eq if match is None else (match & eq)
        kvec = lax.broadcasted_iota(jnp.int32, (K, P), 0)
        cls = jnp.min(jnp.where(match, kvec, KPAD - 1), axis=0, keepdims=True)
        onehot_ref[b] = (
            cls == lax.broadcasted_iota(jnp.int32, (KPAD, P), 0)
        ).astype(jnp.float32)

    def inner_body(*out_blks):
        bi = pl.program_id(0)
        dt = pl.program_id(1)
        pc = pl.program_id(2)
        oh = onehot_ref[bi, :, pl.ds(pc * PC, PC)]
        for s, out_blk in enumerate(out_blks):
            tb = table_ref[pl.ds((dt * NS + s) * DT, DT), :]
            out_blk[0] = lax.dot_general(
                tb, oh, (((1,), (0,)), ((), ())),
                preferred_element_type=jnp.float32)

    pipe = pltpu.emit_pipeline(
        inner_body,
        grid=(B, D // DT // NS, P // PC),
        out_specs=[
            pl.BlockSpec((1, DT, PC),
                         lambda b, j, k, s=s: (b, j * NS + s, k))
            for s in range(NS)
        ],
    )
    pipe(*([out_hbm] * NS))


def kernel(src, colors, feats):
    src_flat = src.reshape(B, 3, P)
    colors_i = colors.astype(jnp.int32)
    table = jnp.zeros((D, KPAD), jnp.float32).at[:, :K].set(feats.T)
    out = pl.pallas_call(
        _outer_body,
        in_specs=[
            pl.BlockSpec(memory_space=pltpu.VMEM),
            pl.BlockSpec(memory_space=pltpu.VMEM),
            pl.BlockSpec(memory_space=pltpu.VMEM),
        ],
        out_specs=pl.BlockSpec(memory_space=pl.ANY),
        out_shape=jax.ShapeDtypeStruct((B, D, P), jnp.float32),
        scratch_shapes=[pltpu.VMEM((B, KPAD, P), jnp.float32)],
    )(src_flat, colors_i, table)
    return out.reshape(B, D, H, W)
